# baseline (speedup 1.0000x reference)
import functools

import jax
import jax.numpy as jnp
from jax import lax
from jax.experimental import pallas as pl
from jax.experimental.pallas import tpu as pltpu
from jax.experimental.pallas import tpu_sc as plsc

N_NODES = 10000
D = 128
N_EDGES = 320000

NC = 2
NS = 16

K = 128
CH0 = 100
CH1 = 60
N_CHUNK_ROWS = NS * (CH0 + CH1)
E_PAD = N_CHUNK_ROWS * K

ROWS_PER_TILE = 640
N_PAD = ROWS_PER_TILE * NS

_mesh = plsc.VectorSubcoreMesh(
    core_axis_name="c", subcore_axis_name="s", num_cores=NC, num_subcores=NS)


@functools.partial(
    pl.kernel,
    out_type=jax.ShapeDtypeStruct((NC, N_PAD, D), jnp.float32),
    mesh=_mesh,
    scratch_types=[
        pltpu.VMEM((K,), jnp.int32),
        pltpu.VMEM((K,), jnp.int32),
        pltpu.VMEM((K, D), jnp.float32),
        pltpu.VMEM_SHARED((N_PAD, D), jnp.float32),
        pltpu.SemaphoreType.DMA,
    ],
)
def _sc_scatter(x_hbm, src_hbm, dst_hbm, out_hbm,
                src_v, dst_v, rows_v, acc, sem):
    c = lax.axis_index("c")
    s = lax.axis_index("s")

    zv = jnp.zeros((16,), jnp.float32)

    def zbody(i, carry):
        rows_v[i // 8, pl.ds((i % 8) * 16, 16)] = zv
        return carry

    lax.fori_loop(0, K * D // 16, zbody, 0)
    row0 = s * ROWS_PER_TILE
    for j in range(ROWS_PER_TILE // K):
        pltpu.sync_copy(rows_v, acc.at[pl.ds(row0 + j * K, K)])
    plsc.subcore_barrier()

    crow = (s * NC + c) * 80

    def body(i, carry):
        pltpu.sync_copy(src_hbm.at[crow + i], src_v)
        pltpu.sync_copy(dst_hbm.at[crow + i], dst_v)
        pltpu.async_copy(x_hbm.at[src_v], rows_v, sem).wait()
        pltpu.sync_copy(rows_v, acc.at[dst_v], add=True)
        return carry

    lax.fori_loop(0, 80, body, 0)
    plsc.subcore_barrier()

    pltpu.sync_copy(acc.at[pl.ds(row0, ROWS_PER_TILE)],
                    out_hbm.at[c, pl.ds(row0, ROWS_PER_TILE)])


def _combine_body(a0_ref, a1_ref, x_ref, w_ref, o_ref):
    t = a0_ref[0] + a1_ref[0] + x_ref[...]
    o_ref[...] = jnp.dot(t, w_ref[...], preferred_element_type=jnp.float32)


_R_BLK = 400


def _combine(agg, x, W):
    return pl.pallas_call(
        _combine_body,
        grid=(N_NODES // _R_BLK,),
        in_specs=[
            pl.BlockSpec((1, _R_BLK, D), lambda i: (0, i, 0)),
            pl.BlockSpec((1, _R_BLK, D), lambda i: (1, i, 0)),
            pl.BlockSpec((_R_BLK, D), lambda i: (i, 0)),
            pl.BlockSpec((D, D), lambda i: (0, 0)),
        ],
        out_specs=pl.BlockSpec((_R_BLK, D), lambda i: (i, 0)),
        out_shape=jax.ShapeDtypeStruct((N_NODES, D), jnp.float32),
    )(agg, agg, x, W)


def kernel(x, edge_index, W):
    src = edge_index[0].astype(jnp.int32)
    dst = edge_index[1].astype(jnp.int32)
    pad = E_PAD - N_EDGES
    src_p = jnp.concatenate([src, jnp.zeros((pad,), jnp.int32)])
    dst_p = jnp.concatenate([dst, jnp.full((pad,), N_NODES, jnp.int32)])
    src_p = src_p.reshape(N_CHUNK_ROWS, K)
    dst_p = dst_p.reshape(N_CHUNK_ROWS, K)
    agg = _sc_scatter(x, src_p, dst_p)
    return _combine(agg, x, W)


# speedup vs baseline: 1.0850x; 1.0850x over previous
import functools

import jax
import jax.numpy as jnp
from jax import lax
from jax.experimental import pallas as pl
from jax.experimental.pallas import tpu as pltpu
from jax.experimental.pallas import tpu_sc as plsc

N_NODES = 10000
D = 128
N_EDGES = 320000

NC = 2
NS = 16

K = 128
CH0 = 100
CH1 = 60
N_CHUNKS = NS * (CH0 + CH1)
E_PAD = N_CHUNKS * K

ROWS_PER_TILE = 640
N_PAD = ROWS_PER_TILE * NS

_mesh = plsc.VectorSubcoreMesh(
    core_axis_name="c", subcore_axis_name="s", num_cores=NC, num_subcores=NS)


@functools.partial(
    pl.kernel,
    out_type=jax.ShapeDtypeStruct((NC, N_PAD, D), jnp.float32),
    mesh=_mesh,
    scratch_types=[
        pltpu.VMEM((K,), jnp.int32),
        pltpu.VMEM((K,), jnp.int32),
        pltpu.VMEM((K, D), jnp.float32),
        pltpu.VMEM_SHARED((N_PAD, D), jnp.float32),
        pltpu.SemaphoreType.DMA,
    ],
)
def _sc_scatter(x_hbm, zeros_hbm, src_hbm, dst_hbm, out_hbm,
                src_v, dst_v, rows_v, acc, sem):
    c = lax.axis_index("c")
    s = lax.axis_index("s")

    row0 = s * ROWS_PER_TILE
    pltpu.sync_copy(zeros_hbm.at[pl.ds(row0, ROWS_PER_TILE)],
                    acc.at[pl.ds(row0, ROWS_PER_TILE)])
    plsc.subcore_barrier()

    nchunks = jnp.where(c == 0, CH0, CH1)
    base = jnp.where(c == 0, s * (CH0 * K), NS * (CH0 * K) + s * (CH1 * K))

    def body(i, carry):
        @pl.when(i < nchunks)
        def _():
            off = base + i * K
            pltpu.sync_copy(src_hbm.at[pl.ds(off, K)], src_v)
            pltpu.sync_copy(dst_hbm.at[pl.ds(off, K)], dst_v)
            pltpu.async_copy(x_hbm.at[src_v], rows_v, sem).wait()
            pltpu.sync_copy(rows_v, acc.at[dst_v], add=True)
        return carry

    lax.fori_loop(0, CH0, body, 0)
    plsc.subcore_barrier()

    pltpu.sync_copy(acc.at[pl.ds(row0, ROWS_PER_TILE)],
                    out_hbm.at[c, pl.ds(row0, ROWS_PER_TILE)])


def _combine_body(a0_ref, a1_ref, x_ref, w_ref, o_ref):
    t = a0_ref[0] + a1_ref[0] + x_ref[...]
    o_ref[...] = jnp.dot(t, w_ref[...], preferred_element_type=jnp.float32)


_R_BLK = 400


def _combine(agg, x, W):
    return pl.pallas_call(
        _combine_body,
        grid=(N_NODES // _R_BLK,),
        in_specs=[
            pl.BlockSpec((1, _R_BLK, D), lambda i: (0, i, 0)),
            pl.BlockSpec((1, _R_BLK, D), lambda i: (1, i, 0)),
            pl.BlockSpec((_R_BLK, D), lambda i: (i, 0)),
            pl.BlockSpec((D, D), lambda i: (0, 0)),
        ],
        out_specs=pl.BlockSpec((_R_BLK, D), lambda i: (i, 0)),
        out_shape=jax.ShapeDtypeStruct((N_NODES, D), jnp.float32),
    )(agg, agg, x, W)


def kernel(x, edge_index, W):
    src = edge_index[0].astype(jnp.int32)
    dst = edge_index[1].astype(jnp.int32)
    pad = E_PAD - N_EDGES
    src_p = jnp.concatenate([src, jnp.zeros((pad,), jnp.int32)])
    dst_p = jnp.concatenate([dst, jnp.full((pad,), N_NODES, jnp.int32)])
    zeros = jnp.zeros((N_PAD, D), jnp.float32)
    agg = _sc_scatter(x, zeros, src_p, dst_p)
    return _combine(agg, x, W)


# speedup vs baseline: 1.4508x; 1.3371x over previous
import functools

import jax
import jax.numpy as jnp
from jax import lax
from jax.experimental import pallas as pl
from jax.experimental.pallas import tpu as pltpu
from jax.experimental.pallas import tpu_sc as plsc

N_NODES = 10000
D = 128
N_EDGES = 320000

NC = 2
NS = 16
NW = NC * NS

K = 128
E_PAD = ((N_EDGES + NW * K - 1) // (NW * K)) * (NW * K)
EDGES_PER_W = E_PAD // NW
CHUNKS_PER_W = EDGES_PER_W // K

ROWS_PER_TILE = 640
N_PAD = ROWS_PER_TILE * NS

_mesh = plsc.VectorSubcoreMesh(
    core_axis_name="c", subcore_axis_name="s", num_cores=NC, num_subcores=NS)


@functools.partial(
    pl.kernel,
    out_type=jax.ShapeDtypeStruct((NC, N_PAD, D), jnp.float32),
    mesh=_mesh,
    scratch_types=[
        pltpu.VMEM((K,), jnp.int32),
        pltpu.VMEM((K,), jnp.int32),
        pltpu.VMEM((K, D), jnp.float32),
        pltpu.VMEM_SHARED((N_PAD, D), jnp.float32),
        pltpu.SemaphoreType.DMA,
    ],
)
def _sc_scatter(x_hbm, zeros_hbm, src_hbm, dst_hbm, out_hbm,
                src_v, dst_v, rows_v, acc, sem):
    c = lax.axis_index("c")
    s = lax.axis_index("s")

    row0 = s * ROWS_PER_TILE
    pltpu.sync_copy(zeros_hbm.at[pl.ds(row0, ROWS_PER_TILE)],
                    acc.at[pl.ds(row0, ROWS_PER_TILE)])
    plsc.subcore_barrier()

    wid = s * NC + c
    base = wid * EDGES_PER_W

    def body(i, carry):
        off = base + i * K
        pltpu.sync_copy(src_hbm.at[pl.ds(off, K)], src_v)
        pltpu.sync_copy(dst_hbm.at[pl.ds(off, K)], dst_v)
        pltpu.async_copy(x_hbm.at[src_v], rows_v, sem).wait()
        pltpu.sync_copy(rows_v, acc.at[dst_v], add=True)
        return carry

    lax.fori_loop(0, CHUNKS_PER_W, body, 0)
    plsc.subcore_barrier()

    pltpu.sync_copy(acc.at[pl.ds(row0, ROWS_PER_TILE)],
                    out_hbm.at[c, pl.ds(row0, ROWS_PER_TILE)])


def _combine_body(a0_ref, a1_ref, x_ref, w_ref, o_ref):
    s = a0_ref[0] + a1_ref[0] + x_ref[...]
    o_ref[...] = jnp.dot(s, w_ref[...], preferred_element_type=jnp.float32)


_R_BLK = 400


def _combine(agg, x, W):
    return pl.pallas_call(
        _combine_body,
        grid=(N_NODES // _R_BLK,),
        in_specs=[
            pl.BlockSpec((1, _R_BLK, D), lambda i: (0, i, 0)),
            pl.BlockSpec((1, _R_BLK, D), lambda i: (1, i, 0)),
            pl.BlockSpec((_R_BLK, D), lambda i: (i, 0)),
            pl.BlockSpec((D, D), lambda i: (0, 0)),
        ],
        out_specs=pl.BlockSpec((_R_BLK, D), lambda i: (i, 0)),
        out_shape=jax.ShapeDtypeStruct((N_NODES, D), jnp.float32),
    )(agg, agg, x, W)


def kernel(x, edge_index, W):
    src = edge_index[0].astype(jnp.int32)
    dst = edge_index[1].astype(jnp.int32)
    pad = E_PAD - N_EDGES
    src_p = jnp.concatenate([src, jnp.zeros((pad,), jnp.int32)])
    dst_p = jnp.concatenate([dst, jnp.full((pad,), N_NODES, jnp.int32)])
    zeros = jnp.zeros((N_PAD, D), jnp.float32)
    agg = _sc_scatter(x, zeros, src_p, dst_p)
    return _combine(agg, x, W)


# speedup vs baseline: 2.2819x; 1.5729x over previous
import functools

import jax
import jax.numpy as jnp
from jax import lax
from jax.experimental import pallas as pl
from jax.experimental.pallas import tpu as pltpu
from jax.experimental.pallas import tpu_sc as plsc

N_NODES = 10000
D = 128
N_EDGES = 320000

NC = 2
NS = 16
NW = NC * NS

K = 128
E_PAD = ((N_EDGES + NW * K - 1) // (NW * K)) * (NW * K)
EDGES_PER_W = E_PAD // NW
CHUNKS_PER_W = EDGES_PER_W // K

ROWS_PER_TILE = 640
N_PAD = ROWS_PER_TILE * NS

_mesh = plsc.VectorSubcoreMesh(
    core_axis_name="c", subcore_axis_name="s", num_cores=NC, num_subcores=NS)


@functools.partial(
    pl.kernel,
    out_type=jax.ShapeDtypeStruct((NC, N_PAD, D), jnp.float32),
    mesh=_mesh,
    scratch_types=[
        pltpu.VMEM((K,), jnp.int32),
        pltpu.VMEM((K,), jnp.int32),
        pltpu.VMEM((K, D), jnp.float32),
        pltpu.VMEM_SHARED((N_PAD, D), jnp.float32),
        pltpu.SemaphoreType.DMA,
    ],
)
def _sc_scatter(x_hbm, zeros_hbm, src_hbm, dst_hbm, out_hbm,
                src_v, dst_v, rows_v, acc, sem):
    c = lax.axis_index("c")
    s = lax.axis_index("s")

    row0 = s * ROWS_PER_TILE
    pltpu.sync_copy(zeros_hbm.at[pl.ds(row0, ROWS_PER_TILE)],
                    acc.at[pl.ds(row0, ROWS_PER_TILE)])
    plsc.subcore_barrier()

    wid = s * NC + c
    base = wid * EDGES_PER_W

    def body(i, carry):
        off = base + i * K
        pltpu.sync_copy(src_hbm.at[pl.ds(off, K)], src_v)
        pltpu.sync_copy(dst_hbm.at[pl.ds(off, K)], dst_v)
        pltpu.async_copy(x_hbm.at[src_v], rows_v, sem).wait()
        pltpu.sync_copy(rows_v, acc.at[dst_v], add=True)
        return carry

    lax.fori_loop(0, CHUNKS_PER_W, body, 0)
    plsc.subcore_barrier()

    pltpu.sync_copy(acc.at[pl.ds(row0, ROWS_PER_TILE)],
                    out_hbm.at[c, pl.ds(row0, ROWS_PER_TILE)])


def _combine_body(a0_ref, a1_ref, x_ref, w_ref, o_ref):
    s = a0_ref[0] + a1_ref[0] + x_ref[...]
    o_ref[...] = jnp.dot(s, w_ref[...], preferred_element_type=jnp.float32)


_R_BLK = 400


def _combine(agg, x, W):
    return pl.pallas_call(
        _combine_body,
        grid=(N_NODES // _R_BLK,),
        in_specs=[
            pl.BlockSpec((1, _R_BLK, D), lambda i: (0, i, 0)),
            pl.BlockSpec((1, _R_BLK, D), lambda i: (1, i, 0)),
            pl.BlockSpec((_R_BLK, D), lambda i: (i, 0)),
            pl.BlockSpec((D, D), lambda i: (0, 0)),
        ],
        out_specs=pl.BlockSpec((_R_BLK, D), lambda i: (i, 0)),
        out_shape=jax.ShapeDtypeStruct((N_NODES, D), jnp.float32),
    )(agg, agg, x, W)


def kernel(x, edge_index, W):
    src = edge_index[0].astype(jnp.int32)
    dst = edge_index[1].astype(jnp.int32)
    pad = E_PAD - N_EDGES
    pad_ar = jnp.arange(pad, dtype=jnp.int32)
    src_p = jnp.concatenate([src, pad_ar % N_NODES])
    dst_p = jnp.concatenate([dst, N_NODES + (pad_ar % (N_PAD - N_NODES))])
    zeros = jnp.zeros((N_PAD, D), jnp.float32)
    agg = _sc_scatter(x, zeros, src_p, dst_p)
    return _combine(agg, x, W)


# speedup vs baseline: 3.0205x; 1.3237x over previous
import functools

import jax
import jax.numpy as jnp
from jax import lax
from jax.experimental import pallas as pl
from jax.experimental.pallas import tpu as pltpu
from jax.experimental.pallas import tpu_sc as plsc

N_NODES = 10000
D = 128
N_EDGES = 320000

NC = 2
NS = 16
NW = NC * NS

K = 128
CHUNKS_PER_W = 80
EDGES_PER_W = CHUNKS_PER_W * K
E_PAD = EDGES_PER_W * NW

ROWS_PER_TILE = 640
N_PAD = ROWS_PER_TILE * NS

_mesh = plsc.VectorSubcoreMesh(
    core_axis_name="c", subcore_axis_name="s", num_cores=NC, num_subcores=NS)


@functools.partial(
    pl.kernel,
    out_type=jax.ShapeDtypeStruct((NC, N_PAD, D), jnp.float32),
    mesh=_mesh,
    scratch_types=[
        pltpu.VMEM((K,), jnp.int32),
        pltpu.VMEM((K,), jnp.int32),
        pltpu.VMEM((K,), jnp.int32),
        pltpu.VMEM((K,), jnp.int32),
        pltpu.VMEM((K, D), jnp.float32),
        pltpu.VMEM((K, D), jnp.float32),
        pltpu.VMEM_SHARED((N_PAD, D), jnp.float32),
        pltpu.SemaphoreType.DMA,
        pltpu.SemaphoreType.DMA,
        pltpu.SemaphoreType.DMA,
        pltpu.SemaphoreType.DMA,
    ],
)
def _sc_scatter(x_hbm, zeros_hbm, src_hbm, dst_hbm, out_hbm,
                sidx0, sidx1, didx0, didx1, rows0, rows1, acc,
                gs0, gs1, ss0, ss1):
    c = lax.axis_index("c")
    s = lax.axis_index("s")

    row0 = s * ROWS_PER_TILE
    pltpu.sync_copy(zeros_hbm.at[pl.ds(row0, ROWS_PER_TILE)],
                    acc.at[pl.ds(row0, ROWS_PER_TILE)])
    plsc.subcore_barrier()

    wid = s * NC + c
    base = wid * EDGES_PER_W

    def load_idx(i, sbuf, dbuf):
        off = base + i * K
        pltpu.sync_copy(src_hbm.at[pl.ds(off, K)], sbuf)
        pltpu.sync_copy(dst_hbm.at[pl.ds(off, K)], dbuf)

    def gather(sbuf, rbuf, sem):
        pltpu.async_copy(x_hbm.at[sbuf], rbuf, sem)

    def wait_gather(sbuf, rbuf, sem):
        pltpu.make_async_copy(x_hbm.at[sbuf], rbuf, sem).wait()

    def scatter(dbuf, rbuf, sem):
        pltpu.async_copy(rbuf, acc.at[dbuf], sem, add=True)

    def wait_scatter(dbuf, rbuf, sem):
        pltpu.make_async_copy(rbuf, acc.at[dbuf], sem).wait()

    load_idx(0, sidx0, didx0)
    gather(sidx0, rows0, gs0)
    load_idx(1, sidx1, didx1)
    gather(sidx1, rows1, gs1)

    def body(p, carry):
        i0 = 2 * p
        wait_gather(sidx0, rows0, gs0)
        scatter(didx0, rows0, ss0)
        wait_gather(sidx1, rows1, gs1)
        wait_scatter(didx0, rows0, ss0)
        load_idx(i0 + 2, sidx0, didx0)
        gather(sidx0, rows0, gs0)
        scatter(didx1, rows1, ss1)
        wait_scatter(didx1, rows1, ss1)
        load_idx(i0 + 3, sidx1, didx1)
        gather(sidx1, rows1, gs1)
        return carry

    lax.fori_loop(0, CHUNKS_PER_W // 2 - 1, body, 0)
    wait_gather(sidx0, rows0, gs0)
    scatter(didx0, rows0, ss0)
    wait_gather(sidx1, rows1, gs1)
    scatter(didx1, rows1, ss1)
    wait_scatter(didx0, rows0, ss0)
    wait_scatter(didx1, rows1, ss1)
    plsc.subcore_barrier()

    pltpu.sync_copy(acc.at[pl.ds(row0, ROWS_PER_TILE)],
                    out_hbm.at[c, pl.ds(row0, ROWS_PER_TILE)])


def _combine_body(a0_ref, a1_ref, x_ref, w_ref, o_ref):
    s = a0_ref[0] + a1_ref[0] + x_ref[...]
    o_ref[...] = jnp.dot(s, w_ref[...], preferred_element_type=jnp.float32)


_R_BLK = 400


def _combine(agg, x, W):
    return pl.pallas_call(
        _combine_body,
        grid=(N_NODES // _R_BLK,),
        in_specs=[
            pl.BlockSpec((1, _R_BLK, D), lambda i: (0, i, 0)),
            pl.BlockSpec((1, _R_BLK, D), lambda i: (1, i, 0)),
            pl.BlockSpec((_R_BLK, D), lambda i: (i, 0)),
            pl.BlockSpec((D, D), lambda i: (0, 0)),
        ],
        out_specs=pl.BlockSpec((_R_BLK, D), lambda i: (i, 0)),
        out_shape=jax.ShapeDtypeStruct((N_NODES, D), jnp.float32),
    )(agg, agg, x, W)


def kernel(x, edge_index, W):
    src = edge_index[0].astype(jnp.int32)
    dst = edge_index[1].astype(jnp.int32)
    pad = E_PAD - N_EDGES
    pad_ar = jnp.arange(pad, dtype=jnp.int32)
    src_p = jnp.concatenate([src, pad_ar % N_NODES])
    dst_p = jnp.concatenate([dst, N_NODES + (pad_ar % (N_PAD - N_NODES))])
    zeros = jnp.zeros((N_PAD, D), jnp.float32)
    agg = _sc_scatter(x, zeros, src_p, dst_p)
    return _combine(agg, x, W)


# speedup vs baseline: 4.1639x; 1.3785x over previous
import functools

import jax
import jax.numpy as jnp
from jax import lax
from jax.experimental import pallas as pl
from jax.experimental.pallas import tpu as pltpu
from jax.experimental.pallas import tpu_sc as plsc

N_NODES = 10000
D = 128
N_EDGES = 320000

NC = 2
NS = 16
NW = NC * NS

K = 128
CHUNKS_PER_W = 80
EDGES_PER_W = CHUNKS_PER_W * K
E_PAD = EDGES_PER_W * NW

ROWS_PER_TILE = 640
N_PAD = ROWS_PER_TILE * NS

_mesh = plsc.VectorSubcoreMesh(
    core_axis_name="c", subcore_axis_name="s", num_cores=NC, num_subcores=NS)


@functools.partial(
    pl.kernel,
    out_type=jax.ShapeDtypeStruct((NC, N_PAD, D), jnp.float32),
    mesh=_mesh,
    scratch_types=[
        pltpu.VMEM((K,), jnp.int32),
        pltpu.VMEM((K,), jnp.int32),
        pltpu.VMEM((K,), jnp.int32),
        pltpu.VMEM((K,), jnp.int32),
        pltpu.VMEM((K, D), jnp.float32),
        pltpu.VMEM((K, D), jnp.float32),
        pltpu.VMEM_SHARED((N_PAD, D), jnp.float32),
        pltpu.SemaphoreType.DMA,
        pltpu.SemaphoreType.DMA,
        pltpu.SemaphoreType.DMA,
        pltpu.SemaphoreType.DMA,
        pltpu.SemaphoreType.DMA,
        pltpu.SemaphoreType.DMA,
        pltpu.SemaphoreType.DMA,
        pltpu.SemaphoreType.DMA,
    ],
)
def _sc_scatter(x_hbm, zeros_hbm, src_hbm, dst_hbm, out_hbm,
                sidx0, sidx1, didx0, didx1, rows0, rows1, acc,
                gs0, gs1, ss0, ss1, is0, is1, id0, id1):
    c = lax.axis_index("c")
    s = lax.axis_index("s")

    row0 = s * ROWS_PER_TILE
    pltpu.sync_copy(zeros_hbm.at[pl.ds(row0, ROWS_PER_TILE)],
                    acc.at[pl.ds(row0, ROWS_PER_TILE)])
    plsc.subcore_barrier()

    wid = s * NC + c
    base = wid * EDGES_PER_W

    def copy_sidx(i, sbuf, sem):
        pltpu.async_copy(src_hbm.at[pl.ds(base + i * K, K)], sbuf, sem)

    def wait_sidx(i, sbuf, sem):
        pltpu.make_async_copy(src_hbm.at[pl.ds(base + i * K, K)], sbuf,
                              sem).wait()

    def copy_didx(i, dbuf, sem):
        pltpu.async_copy(dst_hbm.at[pl.ds(base + i * K, K)], dbuf, sem)

    def wait_didx(i, dbuf, sem):
        pltpu.make_async_copy(dst_hbm.at[pl.ds(base + i * K, K)], dbuf,
                              sem).wait()

    def gather(sbuf, rbuf, sem):
        pltpu.async_copy(x_hbm.at[sbuf], rbuf, sem)

    def wait_gather(sbuf, rbuf, sem):
        pltpu.make_async_copy(x_hbm.at[sbuf], rbuf, sem).wait()

    def scatter(dbuf, rbuf, sem):
        pltpu.async_copy(rbuf, acc.at[dbuf], sem, add=True)

    def wait_scatter(dbuf, rbuf, sem):
        pltpu.make_async_copy(rbuf, acc.at[dbuf], sem).wait()

    pltpu.sync_copy(src_hbm.at[pl.ds(base, K)], sidx0)
    pltpu.sync_copy(src_hbm.at[pl.ds(base + K, K)], sidx1)
    copy_didx(0, didx0, id0)
    copy_didx(1, didx1, id1)
    gather(sidx0, rows0, gs0)
    gather(sidx1, rows1, gs1)

    def body(p, carry):
        i0 = 2 * p
        wait_gather(sidx0, rows0, gs0)
        copy_sidx(i0 + 2, sidx0, is0)
        wait_didx(i0, didx0, id0)
        scatter(didx0, rows0, ss0)
        wait_gather(sidx1, rows1, gs1)
        copy_sidx(i0 + 3, sidx1, is1)
        wait_scatter(didx0, rows0, ss0)
        copy_didx(i0 + 2, didx0, id0)
        wait_sidx(i0 + 2, sidx0, is0)
        gather(sidx0, rows0, gs0)
        wait_didx(i0 + 1, didx1, id1)
        scatter(didx1, rows1, ss1)
        wait_scatter(didx1, rows1, ss1)
        copy_didx(i0 + 3, didx1, id1)
        wait_sidx(i0 + 3, sidx1, is1)
        gather(sidx1, rows1, gs1)
        return carry

    lax.fori_loop(0, CHUNKS_PER_W // 2 - 1, body, 0)
    i_last = CHUNKS_PER_W - 2
    wait_gather(sidx0, rows0, gs0)
    wait_didx(i_last, didx0, id0)
    scatter(didx0, rows0, ss0)
    wait_gather(sidx1, rows1, gs1)
    wait_didx(i_last + 1, didx1, id1)
    scatter(didx1, rows1, ss1)
    wait_scatter(didx0, rows0, ss0)
    wait_scatter(didx1, rows1, ss1)
    plsc.subcore_barrier()

    pltpu.sync_copy(acc.at[pl.ds(row0, ROWS_PER_TILE)],
                    out_hbm.at[c, pl.ds(row0, ROWS_PER_TILE)])


def _combine_body(a0_ref, a1_ref, x_ref, w_ref, o_ref):
    s = a0_ref[0] + a1_ref[0] + x_ref[...]
    o_ref[...] = jnp.dot(s, w_ref[...], preferred_element_type=jnp.float32)


_R_BLK = 400


def _combine(agg, x, W):
    return pl.pallas_call(
        _combine_body,
        grid=(N_NODES // _R_BLK,),
        in_specs=[
            pl.BlockSpec((1, _R_BLK, D), lambda i: (0, i, 0)),
            pl.BlockSpec((1, _R_BLK, D), lambda i: (1, i, 0)),
            pl.BlockSpec((_R_BLK, D), lambda i: (i, 0)),
            pl.BlockSpec((D, D), lambda i: (0, 0)),
        ],
        out_specs=pl.BlockSpec((_R_BLK, D), lambda i: (i, 0)),
        out_shape=jax.ShapeDtypeStruct((N_NODES, D), jnp.float32),
    )(agg, agg, x, W)


def kernel(x, edge_index, W):
    src = edge_index[0].astype(jnp.int32)
    dst = edge_index[1].astype(jnp.int32)
    pad = E_PAD - N_EDGES
    pad_ar = jnp.arange(pad, dtype=jnp.int32)
    src_p = jnp.concatenate([src, pad_ar % N_NODES])
    dst_p = jnp.concatenate([dst, N_NODES + (pad_ar % (N_PAD - N_NODES))])
    zeros = jnp.zeros((N_PAD, D), jnp.float32)
    agg = _sc_scatter(x, zeros, src_p, dst_p)
    return _combine(agg, x, W)
